# baseline (speedup 1.0000x reference)
import jax
import jax.numpy as jnp
from jax.experimental import pallas as pl
from jax.experimental.pallas import tpu as pltpu

N = 256
HIDDEN = 768
C = 128
HEADS = 4
KC = 128
STEPS = HIDDEN // KC


def _leaky_relu(x):
    return jnp.where(x >= 0, x, 0.2 * x)


def _elu(x):
    return jnp.where(x > 0, x, jnp.exp(jnp.minimum(x, 0.0)) - 1.0)


def _attn_layer(h, att_s, att_d):
    a_s = jnp.sum(h * att_s, axis=1, keepdims=True)
    a_d = jnp.sum(h * att_d, axis=1, keepdims=True)
    logits = _leaky_relu(a_s + a_d.T)
    m = jnp.max(logits, axis=0, keepdims=True)
    e = jnp.exp(logits - m)
    s = jnp.sum(e, axis=0, keepdims=True)
    coef = e / (s + 1e-16)
    return jax.lax.dot_general(
        coef, h, (((0,), (0,)), ((), ())),
        preferred_element_type=jnp.float32)


def _gat_kernel(x_ref, w1_ref, as1_ref, ad1_ref, b1_ref,
                w2_ref, as2_ref, ad2_ref, b2_ref, wf_ref, bf_ref,
                out_ref, acc_ref):
    k = pl.program_id(0)

    partial = jnp.dot(x_ref[...], w1_ref[...],
                      preferred_element_type=jnp.float32)

    @pl.when(k == 0)
    def _init():
        acc_ref[...] = partial

    @pl.when(k > 0)
    def _accum():
        acc_ref[...] += partial

    @pl.when(k == STEPS - 1)
    def _rest():
        h1 = acc_ref[...]

        outs = []
        for hd in range(HEADS):
            hh = h1[:, hd * C:(hd + 1) * C]
            outs.append(_attn_layer(hh, as1_ref[hd:hd + 1, :], ad1_ref[hd:hd + 1, :]))
        x1 = jnp.concatenate(outs, axis=1) + b1_ref[...]
        x1 = _elu(x1)

        h2 = jnp.dot(x1, w2_ref[...], preferred_element_type=jnp.float32)
        x2 = _attn_layer(h2, as2_ref[...], ad2_ref[...]) + b2_ref[...]
        x2 = _elu(x2)

        xm = jnp.mean(x2, axis=0, keepdims=True)
        out_ref[...] = jnp.dot(xm, wf_ref[...],
                               preferred_element_type=jnp.float32) + bf_ref[...]


@jax.jit
def kernel(node_feats, W1, att_src1, att_dst1, b1,
           W2, att_src2, att_dst2, b2, Wf, bf):
    out = pl.pallas_call(
        _gat_kernel,
        grid=(STEPS,),
        in_specs=[
            pl.BlockSpec((N, KC), lambda k: (0, k)),
            pl.BlockSpec((KC, HEADS * C), lambda k: (k, 0)),
            pl.BlockSpec((HEADS, C), lambda k: (0, 0)),
            pl.BlockSpec((HEADS, C), lambda k: (0, 0)),
            pl.BlockSpec((1, HEADS * C), lambda k: (0, 0)),
            pl.BlockSpec((HEADS * C, C), lambda k: (0, 0)),
            pl.BlockSpec((1, C), lambda k: (0, 0)),
            pl.BlockSpec((1, C), lambda k: (0, 0)),
            pl.BlockSpec((1, C), lambda k: (0, 0)),
            pl.BlockSpec((C, HIDDEN), lambda k: (0, 0)),
            pl.BlockSpec((1, HIDDEN), lambda k: (0, 0)),
        ],
        out_specs=pl.BlockSpec((1, HIDDEN), lambda k: (0, 0)),
        out_shape=jax.ShapeDtypeStruct((1, HIDDEN), jnp.float32),
        scratch_shapes=[pltpu.VMEM((N, HEADS * C), jnp.float32)],
    )(
        node_feats,
        W1,
        att_src1.reshape(HEADS, C),
        att_dst1.reshape(HEADS, C),
        b1.reshape(1, HEADS * C),
        W2,
        att_src2.reshape(1, C),
        att_dst2.reshape(1, C),
        b2.reshape(1, C),
        Wf,
        bf.reshape(1, HIDDEN),
    )
    return out.reshape(HIDDEN)


# speedup vs baseline: 1.3466x; 1.3466x over previous
import jax
import jax.numpy as jnp
from jax.experimental import pallas as pl

N = 256
HIDDEN = 768
C = 128
HEADS = 4


def _leaky_relu(x):
    return jnp.where(x >= 0, x, 0.2 * x)


def _elu(x):
    return jnp.where(x > 0, x, jnp.exp(jnp.minimum(x, 0.0)) - 1.0)


def _attn_layer(h, att_s, att_d):
    a_s = jnp.sum(h * att_s, axis=1, keepdims=True)
    a_d = jnp.sum(h * att_d, axis=1, keepdims=True)
    logits = _leaky_relu(a_s + a_d.T)
    m = jnp.max(logits, axis=0, keepdims=True)
    e = jnp.exp(logits - m)
    s = jnp.sum(e, axis=0, keepdims=True)
    coef = e / (s + 1e-16)
    return jax.lax.dot_general(
        coef, h, (((0,), (0,)), ((), ())),
        preferred_element_type=jnp.float32)


def _gat_kernel(x_ref, w1_ref, as1_ref, ad1_ref, b1_ref,
                w2_ref, as2_ref, ad2_ref, b2_ref, wf_ref, bf_ref,
                out_ref):
    x = x_ref[...]

    h1 = jnp.dot(x, w1_ref[...], preferred_element_type=jnp.float32)
    outs = []
    for hd in range(HEADS):
        hh = h1[:, hd * C:(hd + 1) * C]
        outs.append(_attn_layer(hh, as1_ref[hd:hd + 1, :], ad1_ref[hd:hd + 1, :]))
    x1 = jnp.concatenate(outs, axis=1) + b1_ref[...]
    x1 = _elu(x1)

    h2 = jnp.dot(x1, w2_ref[...], preferred_element_type=jnp.float32)
    x2 = _attn_layer(h2, as2_ref[...], ad2_ref[...]) + b2_ref[...]
    x2 = _elu(x2)

    xm = jnp.mean(x2, axis=0, keepdims=True)
    out_ref[...] = jnp.dot(xm, wf_ref[...],
                           preferred_element_type=jnp.float32) + bf_ref[...]


@jax.jit
def kernel(node_feats, W1, att_src1, att_dst1, b1,
           W2, att_src2, att_dst2, b2, Wf, bf):
    out = pl.pallas_call(
        _gat_kernel,
        out_shape=jax.ShapeDtypeStruct((1, HIDDEN), jnp.float32),
    )(
        node_feats,
        W1,
        att_src1.reshape(HEADS, C),
        att_dst1.reshape(HEADS, C),
        b1.reshape(1, HEADS * C),
        W2,
        att_src2.reshape(1, C),
        att_dst2.reshape(1, C),
        b2.reshape(1, C),
        Wf,
        bf.reshape(1, HIDDEN),
    )
    return out.reshape(HIDDEN)
